# Initial kernel scaffold; baseline (speedup 1.0000x reference)
import jax
import jax.numpy as jnp
from jax.experimental import pallas as pl


def kernel(x, W_enc, b_enc, codebook, W_dec, b_dec):
    raise NotImplementedError("write your pallas kernel here")


# speedup vs baseline: 5.6842x; 5.6842x over previous
import jax
import jax.numpy as jnp
from jax.experimental import pallas as pl
from jax.experimental.pallas import tpu as pltpu

_B, _C, _L, _D, _K = 16, 256, 4096, 256, 128
_LC = 1024


def _vq_body(x_ref, we_ref, be_ref, cb_ref, wd_ref, bd_ref,
             out_ref, idx_ref, loss_ref, dcb_ref):
    b = pl.program_id(0)
    j = pl.program_id(1)
    first = (b == 0) & (j == 0)

    @pl.when(first)
    def _init():
        dcb_ref[...] = jax.lax.dot_general(
            wd_ref[...], cb_ref[...],
            dimension_numbers=(((1,), (1,)), ((), ()))) + bd_ref[...]

    xb = x_ref[0]
    zT = jnp.dot(we_ref[...], xb) + be_ref[...]
    scores = jnp.dot(cb_ref[...], zT)
    z_sq = jnp.sum(zT * zT, axis=0, keepdims=True)
    cb_sq = jnp.sum(cb_ref[...] * cb_ref[...], axis=1, keepdims=True)
    d2 = z_sq - 2.0 * scores + cb_sq

    mind2 = jnp.min(d2, axis=0)
    iota_k = jax.lax.broadcasted_iota(jnp.int32, (_K, _LC), 0)
    idx = jnp.min(jnp.where(d2 == mind2[None, :], iota_k, _K), axis=0)
    idx_ref[0, 0, :] = idx

    onehot = (iota_k == idx[None, :]).astype(jnp.float32)
    out_ref[0] = jnp.dot(dcb_ref[...], onehot)

    part = jnp.sum(mind2).reshape(1, 1)

    @pl.when(first)
    def _set():
        loss_ref[...] = part

    @pl.when(jnp.logical_not(first))
    def _acc():
        loss_ref[...] += part


def kernel(x, W_enc, b_enc, codebook, W_dec, b_dec):
    grid = (_B, _L // _LC)
    out, idx3, loss_sum = pl.pallas_call(
        _vq_body,
        grid=grid,
        in_specs=[
            pl.BlockSpec((1, _C, _LC), lambda b, j: (b, 0, j)),
            pl.BlockSpec((_D, _C), lambda b, j: (0, 0)),
            pl.BlockSpec((_D, 1), lambda b, j: (0, 0)),
            pl.BlockSpec((_K, _D), lambda b, j: (0, 0)),
            pl.BlockSpec((_C, _D), lambda b, j: (0, 0)),
            pl.BlockSpec((_C, 1), lambda b, j: (0, 0)),
        ],
        out_specs=[
            pl.BlockSpec((1, _C, _LC), lambda b, j: (b, 0, j)),
            pl.BlockSpec((1, 1, _LC), lambda b, j: (b, 0, j)),
            pl.BlockSpec((1, 1), lambda b, j: (0, 0)),
        ],
        out_shape=[
            jax.ShapeDtypeStruct((_B, _C, _L), jnp.float32),
            jax.ShapeDtypeStruct((_B, 1, _L), jnp.int32),
            jax.ShapeDtypeStruct((1, 1), jnp.float32),
        ],
        scratch_shapes=[pltpu.VMEM((_C, _K), jnp.float32)],
    )(x, W_enc, b_enc.reshape(_D, 1), codebook, W_dec, b_dec.reshape(_C, 1))
    indices = idx3.reshape(_B, _L)
    commit_loss = (loss_sum[0, 0] / (_B * _L * _D)).astype(jnp.float32)
    return (out, indices, commit_loss)


# speedup vs baseline: 7.6589x; 1.3474x over previous
import jax
import jax.numpy as jnp
from jax.experimental import pallas as pl
from jax.experimental.pallas import tpu as pltpu

_B, _C, _L, _D, _K = 16, 256, 4096, 256, 128
_LC = 2048


def _vq_body(x_ref, we_ref, be_ref, cb_ref, wd_ref, bd_ref,
             out_ref, idx_ref, loss_ref, dcb_ref):
    b = pl.program_id(0)
    j = pl.program_id(1)
    first = (b == 0) & (j == 0)

    @pl.when(first)
    def _init():
        dcb_ref[...] = jax.lax.dot_general(
            wd_ref[...], cb_ref[...],
            dimension_numbers=(((1,), (1,)), ((), ()))) + bd_ref[...]

    xb = x_ref[0]
    zT = jnp.dot(we_ref[...], xb) + be_ref[...]
    scores = jnp.dot(cb_ref[...], zT)
    z_sq = jnp.sum(zT * zT, axis=0, keepdims=True)
    cb_sq = jnp.sum(cb_ref[...] * cb_ref[...], axis=1, keepdims=True)
    d2 = z_sq - 2.0 * scores + cb_sq

    mind2 = jnp.min(d2, axis=0)
    iota_k = jax.lax.broadcasted_iota(jnp.int32, (_K, _LC), 0)
    idx = jnp.min(jnp.where(d2 == mind2[None, :], iota_k, _K), axis=0)
    idx_ref[0, 0, :] = idx

    onehot = (iota_k == idx[None, :]).astype(jnp.float32)
    out_ref[0] = jnp.dot(dcb_ref[...], onehot)

    part = jnp.sum(mind2).reshape(1, 1)

    @pl.when(first)
    def _set():
        loss_ref[...] = part

    @pl.when(jnp.logical_not(first))
    def _acc():
        loss_ref[...] += part


def kernel(x, W_enc, b_enc, codebook, W_dec, b_dec):
    grid = (_B, _L // _LC)
    out, idx3, loss_sum = pl.pallas_call(
        _vq_body,
        grid=grid,
        in_specs=[
            pl.BlockSpec((1, _C, _LC), lambda b, j: (b, 0, j)),
            pl.BlockSpec((_D, _C), lambda b, j: (0, 0)),
            pl.BlockSpec((_D, 1), lambda b, j: (0, 0)),
            pl.BlockSpec((_K, _D), lambda b, j: (0, 0)),
            pl.BlockSpec((_C, _D), lambda b, j: (0, 0)),
            pl.BlockSpec((_C, 1), lambda b, j: (0, 0)),
        ],
        out_specs=[
            pl.BlockSpec((1, _C, _LC), lambda b, j: (b, 0, j)),
            pl.BlockSpec((1, 1, _LC), lambda b, j: (b, 0, j)),
            pl.BlockSpec((1, 1), lambda b, j: (0, 0)),
        ],
        out_shape=[
            jax.ShapeDtypeStruct((_B, _C, _L), jnp.float32),
            jax.ShapeDtypeStruct((_B, 1, _L), jnp.int32),
            jax.ShapeDtypeStruct((1, 1), jnp.float32),
        ],
        scratch_shapes=[pltpu.VMEM((_C, _K), jnp.float32)],
    )(x, W_enc, b_enc.reshape(_D, 1), codebook, W_dec, b_dec.reshape(_C, 1))
    indices = idx3.reshape(_B, _L)
    commit_loss = (loss_sum[0, 0] / (_B * _L * _D)).astype(jnp.float32)
    return (out, indices, commit_loss)


# speedup vs baseline: 8.9280x; 1.1657x over previous
import jax
import jax.numpy as jnp
from jax.experimental import pallas as pl
from jax.experimental.pallas import tpu as pltpu

_B, _C, _L, _D, _K = 16, 256, 4096, 256, 128
_LC = 4096


def _vq_body(x_ref, we_ref, be_ref, cb_ref, wd_ref, bd_ref,
             out_ref, idx_ref, loss_ref, dcb_ref):
    b = pl.program_id(0)
    j = pl.program_id(1)
    first = (b == 0) & (j == 0)

    @pl.when(first)
    def _init():
        dcb_ref[...] = jax.lax.dot_general(
            wd_ref[...], cb_ref[...],
            dimension_numbers=(((1,), (1,)), ((), ()))) + bd_ref[...]

    xb = x_ref[0]
    zT = jnp.dot(we_ref[...], xb) + be_ref[...]
    scores = jnp.dot(cb_ref[...], zT)
    z_sq = jnp.sum(zT * zT, axis=0, keepdims=True)
    cb_sq = jnp.sum(cb_ref[...] * cb_ref[...], axis=1, keepdims=True)
    d2 = z_sq - 2.0 * scores + cb_sq

    mind2 = jnp.min(d2, axis=0)
    iota_k = jax.lax.broadcasted_iota(jnp.int32, (_K, _LC), 0)
    idx = jnp.min(jnp.where(d2 == mind2[None, :], iota_k, _K), axis=0)
    idx_ref[0, 0, :] = idx

    onehot = (iota_k == idx[None, :]).astype(jnp.float32)
    out_ref[0] = jnp.dot(dcb_ref[...], onehot)

    part = jnp.sum(mind2).reshape(1, 1)

    @pl.when(first)
    def _set():
        loss_ref[...] = part

    @pl.when(jnp.logical_not(first))
    def _acc():
        loss_ref[...] += part


def kernel(x, W_enc, b_enc, codebook, W_dec, b_dec):
    grid = (_B, _L // _LC)
    out, idx3, loss_sum = pl.pallas_call(
        _vq_body,
        grid=grid,
        in_specs=[
            pl.BlockSpec((1, _C, _LC), lambda b, j: (b, 0, j)),
            pl.BlockSpec((_D, _C), lambda b, j: (0, 0)),
            pl.BlockSpec((_D, 1), lambda b, j: (0, 0)),
            pl.BlockSpec((_K, _D), lambda b, j: (0, 0)),
            pl.BlockSpec((_C, _D), lambda b, j: (0, 0)),
            pl.BlockSpec((_C, 1), lambda b, j: (0, 0)),
        ],
        out_specs=[
            pl.BlockSpec((1, _C, _LC), lambda b, j: (b, 0, j)),
            pl.BlockSpec((1, 1, _LC), lambda b, j: (b, 0, j)),
            pl.BlockSpec((1, 1), lambda b, j: (0, 0)),
        ],
        out_shape=[
            jax.ShapeDtypeStruct((_B, _C, _L), jnp.float32),
            jax.ShapeDtypeStruct((_B, 1, _L), jnp.int32),
            jax.ShapeDtypeStruct((1, 1), jnp.float32),
        ],
        scratch_shapes=[pltpu.VMEM((_C, _K), jnp.float32)],
    )(x, W_enc, b_enc.reshape(_D, 1), codebook, W_dec, b_dec.reshape(_C, 1))
    indices = idx3.reshape(_B, _L)
    commit_loss = (loss_sum[0, 0] / (_B * _L * _D)).astype(jnp.float32)
    return (out, indices, commit_loss)
